# baseline (speedup 1.0000x reference)
import functools

import jax
import jax.numpy as jnp
from jax import lax
from jax.experimental import pallas as pl
from jax.experimental.pallas import tpu as pltpu
from jax.experimental.pallas import tpu_sc as plsc

NC = 2
NS = 16
LANES = 16
NW = NC * NS


def _make_pool_kernel(B, L, EMB, chunk):
  bags_per_w = B // NW
  nchunks = bags_per_w // chunk
  nt = 2 * nchunks
  nidx = chunk * L
  nv = EMB // LANES
  inv_l = jnp.float32(1.0 / L)
  mesh = plsc.VectorSubcoreMesh(
      core_axis_name="c", subcore_axis_name="s",
      num_cores=NC, num_subcores=NS)

  def body(idx_hbm, table_hbm, out_hbm, idx_v0, idx_v1, rows0, rows1, outbuf,
           sem0, sem1):
    wid = lax.axis_index("s") * NC + lax.axis_index("c")
    wbase = wid * bags_per_w
    bufs = ((idx_v0, rows0, sem0), (idx_v1, rows1, sem1))

    def idx_off(t):
      s = (t >= nchunks).astype(jnp.int32)
      cc = t - s * nchunks
      return s * (B * L) + (wbase + cc * chunk) * L

    def stage_and_start(t, ib, rb, sm):
      pltpu.sync_copy(idx_hbm.at[pl.ds(idx_off(t), nidx)], ib)
      pltpu.async_copy(table_hbm.at[ib], rb, sm)

    stage_and_start(jnp.int32(0), *bufs[0])

    def pair_body(p, carry):
      for b in range(2):
        t = p * 2 + b
        ib, rb, sm = bufs[b]
        nib, nrb, nsm = bufs[1 - b]

        @pl.when(t + 1 < nt)
        def _():
          stage_and_start(t + 1, nib, nrb, nsm)

        pltpu.make_async_copy(table_hbm.at[ib], rb, sm).wait()

        for i in range(chunk):
          def jb(j, acc, _rb=rb, _i=i):
            r = _i * L + j
            out = []
            for g in range(nv // 2):
              ab = _rb[r, pl.ds(g * 2 * LANES, 2 * LANES)]
              a, b2 = plsc.unpack(ab, format=plsc.PackFormat.INTERLEAVED)
              out.append(acc[2 * g] + a)
              out.append(acc[2 * g + 1] + b2)
            return tuple(out)
          zero = jnp.zeros((LANES,), jnp.float32)
          acc = lax.fori_loop(0, L, jb, (zero,) * nv)
          for v in range(nv):
            outbuf[i, pl.ds(v * LANES, LANES)] = acc[v] * inv_l

        s = (t >= nchunks).astype(jnp.int32)
        cc = t - s * nchunks
        rowbase = wbase + cc * chunk
        pltpu.sync_copy(
            outbuf,
            out_hbm.at[pl.ds(rowbase, chunk), pl.ds(s * EMB, EMB)])
      return carry

    lax.fori_loop(0, nt // 2, pair_body, 0)

  return pl.kernel(
      body,
      out_type=jax.ShapeDtypeStruct((B, 2 * EMB), jnp.float32),
      mesh=mesh,
      scratch_types=[
          pltpu.VMEM((nidx,), jnp.int32),
          pltpu.VMEM((nidx,), jnp.int32),
          pltpu.VMEM((nidx, EMB), jnp.bfloat16),
          pltpu.VMEM((nidx, EMB), jnp.bfloat16),
          pltpu.VMEM((chunk, EMB), jnp.float32),
          pltpu.SemaphoreType.DMA,
          pltpu.SemaphoreType.DMA,
      ],
      compiler_params=pltpu.CompilerParams(
          use_tc_tiling_on_sc=False, needs_layout_passes=False),
  )


def _flatten_table(table):
  V, E = table.shape
  tt = jnp.transpose(table)
  C = 2048
  nb = pl.cdiv(V, C)
  half = C // 2

  def body(t_ref, o_ref):
    tr = jnp.transpose(t_ref[...].astype(jnp.bfloat16))
    o_ref[...] = jnp.concatenate([tr[:half], tr[half:]], axis=1)

  out = pl.pallas_call(
      body,
      grid=(nb,),
      in_specs=[pl.BlockSpec((E, C), lambda i: (0, i))],
      out_specs=pl.BlockSpec((half, 2 * E), lambda i: (i, 0)),
      out_shape=jax.ShapeDtypeStruct((nb * half, 2 * E), jnp.bfloat16),
  )(tt)
  return out.reshape(nb * C, E)


def _permute_idx(idx):
  l = idx & 2047
  return (idx - l) + ((l & 1023) << 1) + (l >> 10)


def _mlp(pooled, w1, b1, w2, b2, w3, b3):
  B, D = pooled.shape
  R = 2048

  def body(x_ref, w1_ref, b1_ref, w2_ref, b2_ref, w3_ref, b3_ref, o_ref):
    x = x_ref[...]
    h = jnp.maximum(
        jnp.dot(x, w1_ref[...], preferred_element_type=jnp.float32)
        + b1_ref[...], 0.0)
    h = jnp.maximum(
        jnp.dot(h, w2_ref[...], preferred_element_type=jnp.float32)
        + b2_ref[...], 0.0)
    o_ref[...] = (
        jnp.dot(h, w3_ref[...], preferred_element_type=jnp.float32)
        + b3_ref[...])

  wspec = pl.BlockSpec((D, D), lambda i: (0, 0))
  bspec = pl.BlockSpec((1, D), lambda i: (0, 0))
  return pl.pallas_call(
      body,
      grid=(B // R,),
      in_specs=[
          pl.BlockSpec((R, D), lambda i: (i, 0)),
          wspec, bspec, wspec, bspec, wspec, bspec,
      ],
      out_specs=pl.BlockSpec((R, D), lambda i: (i, 0)),
      out_shape=jax.ShapeDtypeStruct((B, D), jnp.float32),
  )(pooled, w1, b1, w2, b2, w3, b3)


def _pad_to(x, shape):
  pads = [(0, t - s) for s, t in zip(x.shape, shape)]
  return jnp.pad(x, pads)


def kernel(data_hypo, length_hypo, data_prem, length_prem, table,
           W1, b1, W2, b2, W3, b3):
  B, L = data_hypo.shape
  EMB = table.shape[1]
  D = 2 * EMB

  idx_h = _permute_idx(jnp.reshape(data_hypo, (-1,)).astype(jnp.int32))
  idx_p = _permute_idx(jnp.reshape(data_prem, (-1,)).astype(jnp.int32))
  idx_all = jnp.concatenate([idx_h, idx_p])

  table_b, idx_all = lax.optimization_barrier((table, idx_all))

  table_lin = _flatten_table(table_b)
  pooled = _make_pool_kernel(B, L, EMB, chunk=8)(idx_all, table_lin)

  perm64 = []
  for g in range(EMB // 32):
    perm64 += [32 * g + 2 * k for k in range(16)]
    perm64 += [32 * g + 2 * k + 1 for k in range(16)]
  perm128 = perm64 + [EMB + p for p in perm64]
  W1 = W1[jnp.asarray(perm128), :]

  w1 = _pad_to(W1, (D, D))
  w2 = _pad_to(W2, (D, D))
  w3 = _pad_to(W3, (D, D))
  b1p = _pad_to(b1.reshape(1, -1), (1, D))
  b2p = _pad_to(b2.reshape(1, -1), (1, D))
  b3p = _pad_to(b3.reshape(1, -1), (1, D))

  y = _mlp(pooled, w1, b1p, w2, b2p, w3, b3p)
  return y[:, :W3.shape[1]]


# speedup vs baseline: 1.3903x; 1.3903x over previous
import functools

import jax
import jax.numpy as jnp
from jax import lax
from jax.experimental import pallas as pl
from jax.experimental.pallas import tpu as pltpu
from jax.experimental.pallas import tpu_sc as plsc

NC = 2
NS = 16
LANES = 16
NW = NC * NS


def _make_pool_kernel(B, L, EMB, chunk):
  bags_per_w = B // NW
  nchunks = bags_per_w // chunk
  nt = 2 * nchunks
  nidx = chunk * L
  nv = EMB // LANES
  inv_l = jnp.float32(1.0 / L)
  mesh = plsc.VectorSubcoreMesh(
      core_axis_name="c", subcore_axis_name="s",
      num_cores=NC, num_subcores=NS)

  def body(idx_hbm, table_hbm, out_hbm, idx_v0, idx_v1, rows0, rows1, outbuf,
           sem0, sem1):
    wid = lax.axis_index("s") * NC + lax.axis_index("c")
    wbase = wid * bags_per_w
    bufs = ((idx_v0, rows0, sem0), (idx_v1, rows1, sem1))

    def idx_off(t):
      s = (t >= nchunks).astype(jnp.int32)
      cc = t - s * nchunks
      return s * (B * L) + (wbase + cc * chunk) * L

    def stage_and_start(t, ib, rb, sm):
      pltpu.sync_copy(idx_hbm.at[pl.ds(idx_off(t), nidx)], ib)
      pltpu.async_copy(table_hbm.at[ib], rb, sm)

    stage_and_start(jnp.int32(0), *bufs[0])

    def pair_body(p, carry):
      for b in range(2):
        t = p * 2 + b
        ib, rb, sm = bufs[b]
        nib, nrb, nsm = bufs[1 - b]

        @pl.when(t + 1 < nt)
        def _():
          stage_and_start(t + 1, nib, nrb, nsm)

        pltpu.make_async_copy(table_hbm.at[ib], rb, sm).wait()

        for i in range(chunk):
          def jb(j, acc, _rb=rb, _i=i):
            r = _i * L + j
            out = []
            for g in range(nv // 2):
              wv = _rb[r, pl.ds(g * LANES, LANES)]
              ab = plsc.bitcast(wv, jnp.bfloat16)
              a, b2 = plsc.unpack(ab, format=plsc.PackFormat.INTERLEAVED)
              out.append(acc[2 * g] + a)
              out.append(acc[2 * g + 1] + b2)
            return tuple(out)
          zero = jnp.zeros((LANES,), jnp.float32)
          acc = lax.fori_loop(0, L, jb, (zero,) * nv)
          for v in range(nv):
            outbuf[i, pl.ds(v * LANES, LANES)] = acc[v] * inv_l

        s = (t >= nchunks).astype(jnp.int32)
        cc = t - s * nchunks
        rowbase = wbase + cc * chunk
        pltpu.sync_copy(
            outbuf,
            out_hbm.at[pl.ds(rowbase, chunk), pl.ds(s * EMB, EMB)])
      return carry

    lax.fori_loop(0, nt // 2, pair_body, 0)

  return pl.kernel(
      body,
      out_type=jax.ShapeDtypeStruct((B, 2 * EMB), jnp.float32),
      mesh=mesh,
      scratch_types=[
          pltpu.VMEM((nidx,), jnp.int32),
          pltpu.VMEM((nidx,), jnp.int32),
          pltpu.VMEM((nidx, EMB // 2), jnp.float32),
          pltpu.VMEM((nidx, EMB // 2), jnp.float32),
          pltpu.VMEM((chunk, EMB), jnp.float32),
          pltpu.SemaphoreType.DMA,
          pltpu.SemaphoreType.DMA,
      ],
      compiler_params=pltpu.CompilerParams(
          use_tc_tiling_on_sc=False, needs_layout_passes=False),
  )


def _flatten_table(table):
  V, E = table.shape
  tt = jnp.transpose(table)
  C = 2048
  Cq = C // 4
  nb = pl.cdiv(V, C)

  def body(t_ref, o_ref):
    tr = jnp.transpose(t_ref[...])
    u = jax.lax.bitcast_convert_type(tr, jnp.uint32)
    r = u + jnp.uint32(0x7FFF) + ((u >> 16) & jnp.uint32(1))
    lo = r[:, :E // 2] >> 16
    hi = r[:, E // 2:] & jnp.uint32(0xFFFF0000)
    w = lo | hi
    packed = jnp.concatenate(
        [w[i * Cq:(i + 1) * Cq] for i in range(4)], axis=1)
    o_ref[...] = jax.lax.bitcast_convert_type(packed, jnp.float32)

  out = pl.pallas_call(
      body,
      grid=(nb,),
      in_specs=[pl.BlockSpec((E, C), lambda i: (0, i))],
      out_specs=pl.BlockSpec((Cq, 2 * E), lambda i: (i, 0)),
      out_shape=jax.ShapeDtypeStruct((nb * Cq, 2 * E), jnp.float32),
  )(tt)
  return out.reshape(nb * C, E // 2)


def _permute_idx(idx):
  l = idx & 2047
  return (idx - l) + ((l & 511) << 2) + (l >> 9)


def _mlp(pooled, w1, b1, w2, b2, w3, b3):
  B, D = pooled.shape
  R = 2048

  def body(x_ref, w1_ref, b1_ref, w2_ref, b2_ref, w3_ref, b3_ref, o_ref):
    x = x_ref[...]
    h = jnp.maximum(
        jnp.dot(x, w1_ref[...], preferred_element_type=jnp.float32)
        + b1_ref[...], 0.0)
    h = jnp.maximum(
        jnp.dot(h, w2_ref[...], preferred_element_type=jnp.float32)
        + b2_ref[...], 0.0)
    o_ref[...] = (
        jnp.dot(h, w3_ref[...], preferred_element_type=jnp.float32)
        + b3_ref[...])

  wspec = pl.BlockSpec((D, D), lambda i: (0, 0))
  bspec = pl.BlockSpec((1, D), lambda i: (0, 0))
  return pl.pallas_call(
      body,
      grid=(B // R,),
      in_specs=[
          pl.BlockSpec((R, D), lambda i: (i, 0)),
          wspec, bspec, wspec, bspec, wspec, bspec,
      ],
      out_specs=pl.BlockSpec((R, D), lambda i: (i, 0)),
      out_shape=jax.ShapeDtypeStruct((B, D), jnp.float32),
  )(pooled, w1, b1, w2, b2, w3, b3)


def _pad_to(x, shape):
  pads = [(0, t - s) for s, t in zip(x.shape, shape)]
  return jnp.pad(x, pads)


def kernel(data_hypo, length_hypo, data_prem, length_prem, table,
           W1, b1, W2, b2, W3, b3):
  B, L = data_hypo.shape
  EMB = table.shape[1]
  D = 2 * EMB

  idx_h = _permute_idx(jnp.reshape(data_hypo, (-1,)).astype(jnp.int32))
  idx_p = _permute_idx(jnp.reshape(data_prem, (-1,)).astype(jnp.int32))
  idx_all = jnp.concatenate([idx_h, idx_p])

  table_b, idx_all = lax.optimization_barrier((table, idx_all))

  table_lin = _flatten_table(table_b)
  pooled = _make_pool_kernel(B, L, EMB, chunk=8)(idx_all, table_lin)

  h = EMB // 4
  perm64 = (list(range(0, h)) + list(range(2 * h, 3 * h))
            + list(range(h, 2 * h)) + list(range(3 * h, 4 * h)))
  perm128 = perm64 + [EMB + p for p in perm64]
  W1 = W1[jnp.asarray(perm128), :]

  w1 = _pad_to(W1, (D, D))
  w2 = _pad_to(W2, (D, D))
  w3 = _pad_to(W3, (D, D))
  b1p = _pad_to(b1.reshape(1, -1), (1, D))
  b2p = _pad_to(b2.reshape(1, -1), (1, D))
  b3p = _pad_to(b3.reshape(1, -1), (1, D))

  y = _mlp(pooled, w1, b1p, w2, b2p, w3, b3p)
  return y[:, :W3.shape[1]]


# speedup vs baseline: 1.4891x; 1.0710x over previous
import functools

import jax
import jax.numpy as jnp
from jax import lax
from jax.experimental import pallas as pl
from jax.experimental.pallas import tpu as pltpu
from jax.experimental.pallas import tpu_sc as plsc

NC = 2
NS = 16
LANES = 16
NW = NC * NS


def _make_pool_kernel(B, L, EMB, chunk):
  bags_per_w = B // NW
  nchunks = bags_per_w // chunk
  nt = 2 * nchunks
  nidx = chunk * L
  nv = EMB // LANES
  inv_l = jnp.float32(1.0 / L)
  mesh = plsc.VectorSubcoreMesh(
      core_axis_name="c", subcore_axis_name="s",
      num_cores=NC, num_subcores=NS)

  def body(idx_hbm, table_hbm, out_hbm, idx_v0, idx_v1, rows0, rows1, outbuf,
           sem0, sem1):
    wid = lax.axis_index("s") * NC + lax.axis_index("c")
    wbase = wid * bags_per_w
    bufs = ((idx_v0, rows0, sem0), (idx_v1, rows1, sem1))

    def idx_off(t):
      s = (t >= nchunks).astype(jnp.int32)
      cc = t - s * nchunks
      return s * (B * L) + (wbase + cc * chunk) * L

    def stage_and_start(t, ib, rb, sm):
      pltpu.sync_copy(idx_hbm.at[pl.ds(idx_off(t), nidx)], ib)
      pltpu.async_copy(table_hbm.at[ib], rb, sm)

    stage_and_start(jnp.int32(0), *bufs[0])

    def pair_body(p, carry):
      for b in range(2):
        t = p * 2 + b
        ib, rb, sm = bufs[b]
        nib, nrb, nsm = bufs[1 - b]

        @pl.when(t + 1 < nt)
        def _():
          stage_and_start(t + 1, nib, nrb, nsm)

        pltpu.make_async_copy(table_hbm.at[ib], rb, sm).wait()

        for i in range(chunk):
          def jb(j, acc, _rb=rb, _i=i):
            r = _i * L + j
            return tuple(acc[v] + _rb[r, pl.ds(v * LANES, LANES)]
                         for v in range(nv))
          zero = jnp.zeros((LANES,), jnp.float32)
          acc = lax.fori_loop(0, L, jb, (zero,) * nv)
          for v in range(nv):
            outbuf[i, pl.ds(v * LANES, LANES)] = acc[v] * inv_l

        s = (t >= nchunks).astype(jnp.int32)
        cc = t - s * nchunks
        rowbase = wbase + cc * chunk
        pltpu.sync_copy(
            outbuf,
            out_hbm.at[pl.ds(rowbase, chunk), pl.ds(s * EMB, EMB)])
      return carry

    lax.fori_loop(0, nt // 2, pair_body, 0)

  return pl.kernel(
      body,
      out_type=jax.ShapeDtypeStruct((B, 2 * EMB), jnp.float32),
      mesh=mesh,
      scratch_types=[
          pltpu.VMEM((nidx,), jnp.int32),
          pltpu.VMEM((nidx,), jnp.int32),
          pltpu.VMEM((nidx, EMB), jnp.float32),
          pltpu.VMEM((nidx, EMB), jnp.float32),
          pltpu.VMEM((chunk, EMB), jnp.float32),
          pltpu.SemaphoreType.DMA,
          pltpu.SemaphoreType.DMA,
      ],
      compiler_params=pltpu.CompilerParams(use_tc_tiling_on_sc=False),
  )


def _flatten_table(table):
  V, E = table.shape
  tt = jnp.transpose(table)
  C = 2048
  nb = pl.cdiv(V, C)
  half = C // 2

  def body(t_ref, o_ref):
    tr = jnp.transpose(t_ref[...])
    o_ref[...] = jnp.concatenate([tr[:half], tr[half:]], axis=1)

  out = pl.pallas_call(
      body,
      grid=(nb,),
      in_specs=[pl.BlockSpec((E, C), lambda i: (0, i))],
      out_specs=pl.BlockSpec((half, 2 * E), lambda i: (i, 0)),
      out_shape=jax.ShapeDtypeStruct((nb * half, 2 * E), jnp.float32),
  )(tt)
  return out.reshape(nb * C, E)


def _permute_idx(idx):
  l = idx & 2047
  return (idx - l) + ((l & 1023) << 1) + (l >> 10)


def _mlp(pooled, w1, b1, w2, b2, w3, b3):
  B, D = pooled.shape
  R = 2048

  def body(x_ref, w1_ref, b1_ref, w2_ref, b2_ref, w3_ref, b3_ref, o_ref):
    x = x_ref[...]
    h = jnp.maximum(
        jnp.dot(x, w1_ref[...], preferred_element_type=jnp.float32)
        + b1_ref[...], 0.0)
    h = jnp.maximum(
        jnp.dot(h, w2_ref[...], preferred_element_type=jnp.float32)
        + b2_ref[...], 0.0)
    o_ref[...] = (
        jnp.dot(h, w3_ref[...], preferred_element_type=jnp.float32)
        + b3_ref[...])

  wspec = pl.BlockSpec((D, D), lambda i: (0, 0))
  bspec = pl.BlockSpec((1, D), lambda i: (0, 0))
  return pl.pallas_call(
      body,
      grid=(B // R,),
      in_specs=[
          pl.BlockSpec((R, D), lambda i: (i, 0)),
          wspec, bspec, wspec, bspec, wspec, bspec,
      ],
      out_specs=pl.BlockSpec((R, D), lambda i: (i, 0)),
      out_shape=jax.ShapeDtypeStruct((B, D), jnp.float32),
  )(pooled, w1, b1, w2, b2, w3, b3)


def _pad_to(x, shape):
  pads = [(0, t - s) for s, t in zip(x.shape, shape)]
  return jnp.pad(x, pads)


def kernel(data_hypo, length_hypo, data_prem, length_prem, table,
           W1, b1, W2, b2, W3, b3):
  B, L = data_hypo.shape
  EMB = table.shape[1]
  D = 2 * EMB

  idx_h = _permute_idx(jnp.reshape(data_hypo, (-1,)).astype(jnp.int32))
  idx_p = _permute_idx(jnp.reshape(data_prem, (-1,)).astype(jnp.int32))
  idx_all = jnp.concatenate([idx_h, idx_p])

  table_b, idx_all = lax.optimization_barrier((table, idx_all))

  table_lin = _flatten_table(table_b)
  pooled = _make_pool_kernel(B, L, EMB, chunk=8)(idx_all, table_lin)

  w1 = _pad_to(W1, (D, D))
  w2 = _pad_to(W2, (D, D))
  w3 = _pad_to(W3, (D, D))
  b1p = _pad_to(b1.reshape(1, -1), (1, D))
  b2p = _pad_to(b2.reshape(1, -1), (1, D))
  b3p = _pad_to(b3.reshape(1, -1), (1, D))

  y = _mlp(pooled, w1, b1p, w2, b2p, w3, b3p)
  return y[:, :W3.shape[1]]
